# baseline (speedup 1.0000x reference)
import functools

import jax
import jax.numpy as jnp
from jax import lax
from jax.experimental import pallas as pl
from jax.experimental.pallas import tpu as pltpu
from jax.experimental.pallas import tpu_sc as plsc

DIM = 64
NC = 2
NS = 16
L = 16
NW = NC * NS
CHUNK = 128


def _rsqrt(x):
    i = plsc.bitcast(x, jnp.int32)
    i = jnp.int32(0x5F3759DF) - lax.shift_right_logical(i, 1)
    y = plsc.bitcast(i, jnp.float32)
    for _ in range(3):
        y = y * (1.5 - 0.5 * x * y * y)
    return y


def _inv_norm(ss):
    n = ss * _rsqrt(ss)
    return 1.0 / jnp.maximum(n, 1e-12)


def _make_kernel(B):
    bpw = B // NW
    nchunks = bpw // CHUNK
    groups = CHUNK // L
    mesh = plsc.VectorSubcoreMesh(
        core_axis_name="c", subcore_axis_name="s", num_cores=NC,
        num_subcores=NS)

    @functools.partial(
        pl.kernel,
        out_type=jax.ShapeDtypeStruct((B,), jnp.float32),
        mesh=mesh,
        compiler_params=pltpu.CompilerParams(
            needs_layout_passes=False, use_tc_tiling_on_sc=False),
        scratch_types=[
            pltpu.VMEM((CHUNK,), jnp.int32),
            pltpu.VMEM((CHUNK,), jnp.int32),
            pltpu.VMEM((CHUNK,), jnp.int32),
            pltpu.VMEM((CHUNK, DIM), jnp.float32),
            pltpu.VMEM((CHUNK, DIM), jnp.float32),
            pltpu.VMEM((CHUNK, DIM), jnp.float32),
            pltpu.VMEM((bpw,), jnp.float32),
            pltpu.SemaphoreType.DMA,
        ],
    )
    def k(h_hbm, r_hbm, t_hbm, ent_hbm, rel_hbm, out_hbm,
          hidx, tidx, ridx, hrows, trows, rrows, outv, sem):
        wid = lax.axis_index("s") * NC + lax.axis_index("c")
        base = wid * bpw
        iota = lax.iota(jnp.int32, L)

        def chunk_body(c, carry):
            off = base + c * CHUNK
            pltpu.sync_copy(h_hbm.at[pl.ds(off, CHUNK)], hidx)
            pltpu.sync_copy(t_hbm.at[pl.ds(off, CHUNK)], tidx)
            pltpu.sync_copy(r_hbm.at[pl.ds(off, CHUNK)], ridx)
            ch = pltpu.async_copy(ent_hbm.at[hidx], hrows, sem)
            ct = pltpu.async_copy(ent_hbm.at[tidx], trows, sem)
            cr = pltpu.async_copy(rel_hbm.at[ridx], rrows, sem)
            ch.wait()
            ct.wait()
            cr.wait()

            def group_body(g, carry2):
                rid = iota + g * L
                zero = jnp.zeros((L,), jnp.float32)
                ssh, sst, ssr = zero, zero, zero
                dht, dhr, dtr = zero, zero, zero
                for d in range(DIM):
                    col = jnp.full((L,), d, jnp.int32)
                    hv = plsc.load_gather(hrows, [rid, col])
                    tv = plsc.load_gather(trows, [rid, col])
                    rv = plsc.load_gather(rrows, [rid, col])
                    ssh += hv * hv
                    sst += tv * tv
                    ssr += rv * rv
                    dht += hv * tv
                    dhr += hv * rv
                    dtr += tv * rv
                a = _inv_norm(ssh)
                b = _inv_norm(sst)
                cc = _inv_norm(ssr)
                q = (ssh * a * a + sst * b * b + ssr * cc * cc
                     - 2.0 * ((a * b) * dht - (a * cc) * dhr + (b * cc) * dtr))
                qm = jnp.maximum(q, 0.0)
                outv[pl.ds(c * CHUNK + g * L, L)] = qm * _rsqrt(qm)
                return carry2

            return lax.fori_loop(0, groups, group_body, carry)

        lax.fori_loop(0, nchunks, chunk_body, 0)
        pltpu.sync_copy(outv, out_hbm.at[pl.ds(base, bpw)])

    return k


def kernel(h, r, t, emb_entity, emb_relation):
    h = h.astype(jnp.int32)
    r = r.astype(jnp.int32)
    t = t.astype(jnp.int32)
    return _make_kernel(h.shape[0])(h, r, t, emb_entity, emb_relation)


# speedup vs baseline: 2.0951x; 2.0951x over previous
import functools

import jax
import jax.numpy as jnp
from jax import lax
from jax.experimental import pallas as pl
from jax.experimental.pallas import tpu as pltpu
from jax.experimental.pallas import tpu_sc as plsc

DIM = 64
TR = 8
NC = 2
NS = 16
L = 16
NW = NC * NS
CHUNK = 32


def _rsqrt(x):
    i = plsc.bitcast(x, jnp.int32)
    i = jnp.int32(0x5F3759DF) - lax.shift_right_logical(i, 1)
    y = plsc.bitcast(i, jnp.float32)
    for _ in range(3):
        y = y * (1.5 - 0.5 * x * y * y)
    return y


def _inv_norm(ss):
    n = ss * _rsqrt(ss)
    return 1.0 / jnp.maximum(n, 1e-12)


def _make_kernel(B):
    bpw = B // NW
    nchunks = bpw // CHUNK
    groups = CHUNK // L
    mesh = plsc.VectorSubcoreMesh(
        core_axis_name="c", subcore_axis_name="s", num_cores=NC,
        num_subcores=NS)

    @functools.partial(
        pl.kernel,
        out_type=jax.ShapeDtypeStruct((B,), jnp.float32),
        mesh=mesh,
        compiler_params=pltpu.CompilerParams(
            needs_layout_passes=False, use_tc_tiling_on_sc=True),
        scratch_types=[
            pltpu.VMEM((CHUNK,), jnp.int32),
            pltpu.VMEM((CHUNK,), jnp.int32),
            pltpu.VMEM((CHUNK,), jnp.int32),
            pltpu.VMEM((CHUNK, DIM), jnp.float32),
            pltpu.VMEM((CHUNK, DIM), jnp.float32),
            pltpu.VMEM((CHUNK, DIM), jnp.float32),
            pltpu.VMEM((bpw,), jnp.float32),
            pltpu.SemaphoreType.DMA,
        ],
    )
    def k(h_hbm, r_hbm, t_hbm, ent_hbm, rel_hbm, out_hbm,
          hidx, tidx, ridx, hrows, trows, rrows,
          outv, sem):
        wid = lax.axis_index("s") * NC + lax.axis_index("c")
        base = wid * bpw
        iota = lax.iota(jnp.int32, L)

        def chunk_body(c, carry):
            off = base + c * CHUNK
            pltpu.sync_copy(h_hbm.at[pl.ds(off, CHUNK)], hidx)
            pltpu.sync_copy(t_hbm.at[pl.ds(off, CHUNK)], tidx)
            pltpu.sync_copy(r_hbm.at[pl.ds(off, CHUNK)], ridx)
            copies = []
            for s in range(CHUNK // L):
                hv16 = hidx[pl.ds(s * L, L)]
                tv16 = tidx[pl.ds(s * L, L)]
                rv16 = ridx[pl.ds(s * L, L)]
                for jl in range(L):
                    j = s * L + jl
                    hj = hv16[jl]
                    tj = tv16[jl]
                    rj = rv16[jl]
                    copies.append(pltpu.async_copy(
                        ent_hbm.at[hj >> 3, hj & 7], hrows.at[j], sem))
                    copies.append(pltpu.async_copy(
                        ent_hbm.at[tj >> 3, tj & 7], trows.at[j], sem))
                    copies.append(pltpu.async_copy(
                        rel_hbm.at[rj >> 3, rj & 7], rrows.at[j], sem))
            for cp in copies:
                cp.wait()

            def group_body(g, carry2):
                jj = iota + g * L
                zero = jnp.zeros((L,), jnp.float32)
                ssh, sst, ssr = zero, zero, zero
                dht, dhr, dtr = zero, zero, zero
                for d in range(DIM):
                    col = jnp.full((L,), d, jnp.int32)
                    hv = plsc.load_gather(hrows, [jj, col])
                    tv = plsc.load_gather(trows, [jj, col])
                    rv = plsc.load_gather(rrows, [jj, col])
                    ssh += hv * hv
                    sst += tv * tv
                    ssr += rv * rv
                    dht += hv * tv
                    dhr += hv * rv
                    dtr += tv * rv
                a = _inv_norm(ssh)
                b = _inv_norm(sst)
                cc = _inv_norm(ssr)
                q = (ssh * a * a + sst * b * b + ssr * cc * cc
                     - 2.0 * ((a * b) * dht - (a * cc) * dhr + (b * cc) * dtr))
                qm = jnp.maximum(q, 0.0)
                outv[pl.ds(c * CHUNK + g * L, L)] = qm * _rsqrt(qm)
                return carry2

            return lax.fori_loop(0, groups, group_body, carry)

        lax.fori_loop(0, nchunks, chunk_body, 0)
        pltpu.sync_copy(outv, out_hbm.at[pl.ds(base, bpw)])

    return k


def kernel(h, r, t, emb_entity, emb_relation):
    h = h.astype(jnp.int32)
    r = r.astype(jnp.int32)
    t = t.astype(jnp.int32)
    ent = emb_entity.reshape(emb_entity.shape[0] // TR, TR, DIM)
    rel = emb_relation.reshape(emb_relation.shape[0] // TR, TR, DIM)
    return _make_kernel(h.shape[0])(h, r, t, ent, rel)
